# Initial kernel scaffold; baseline (speedup 1.0000x reference)
import jax
import jax.numpy as jnp
from jax.experimental import pallas as pl


def kernel(action, centroids):
    raise NotImplementedError("write your pallas kernel here")


# speedup vs baseline: 1.3516x; 1.3516x over previous
import jax
import jax.numpy as jnp
from jax.experimental import pallas as pl


_R = 256


def _vq_body(a_ref, c_ref, bins_ref, res_ref):
    a = a_ref[...]
    c = c_ref[...]
    an = jnp.sum(a * a, axis=1, keepdims=True)
    cn = jnp.sum(c * c, axis=1)
    dots = jax.lax.dot_general(
        a, c, (((1,), (1,)), ((), ())),
        preferred_element_type=jnp.float32)
    sq = (an - 2.0 * dots) + cn[None, :]
    sq = jnp.maximum(sq, 0.0)
    idx = jnp.argmin(sq, axis=1).astype(jnp.int32)
    bins_ref[...] = idx
    k_iota = jax.lax.broadcasted_iota(jnp.int32, sq.shape, 1)
    onehot = (k_iota == idx[:, None]).astype(jnp.float32)
    center = jax.lax.dot_general(
        onehot, c, (((1,), (0,)), ((), ())),
        preferred_element_type=jnp.float32)
    res_ref[...] = a - center


def kernel(action, centroids):
    B, T, D = action.shape
    K = centroids.shape[0]
    N = B * T
    a_flat = action.reshape(N, D)
    grid = (N // _R,)
    bins, res = pl.pallas_call(
        _vq_body,
        grid=grid,
        in_specs=[
            pl.BlockSpec((_R, D), lambda i: (i, 0)),
            pl.BlockSpec((K, D), lambda i: (0, 0)),
        ],
        out_specs=[
            pl.BlockSpec((_R,), lambda i: (i,)),
            pl.BlockSpec((_R, D), lambda i: (i, 0)),
        ],
        out_shape=[
            jax.ShapeDtypeStruct((N,), jnp.int32),
            jax.ShapeDtypeStruct((N, D), jnp.float32),
        ],
    )(a_flat, centroids)
    return (bins.reshape(B, T, 1).astype(jnp.int64),
            res.reshape(B, T, D))


# speedup vs baseline: 1.5116x; 1.1184x over previous
import jax
import jax.numpy as jnp
from jax.experimental import pallas as pl
from jax.experimental.pallas import tpu as pltpu


_R = 256


def _vq_body(a_ref, c_ref, bins_ref, res_ref, cnt_ref, chi_ref, clo_ref):
    K = c_ref.shape[0]

    @pl.when(pl.program_id(0) == 0)
    def _precompute():
        c = c_ref[...]
        chi = c.astype(jnp.bfloat16)
        chi_ref[...] = chi
        clo_ref[...] = (c - chi.astype(jnp.float32)).astype(jnp.bfloat16)
        cn = jnp.sum(c * c, axis=1)
        cnt_ref[...] = cn[None, :]

    a = a_ref[...]
    c = c_ref[...]
    dots = jax.lax.dot_general(
        a, c, (((1,), (1,)), ((), ())),
        preferred_element_type=jnp.float32)
    score = (-2.0) * dots + cnt_ref[...]
    m = jnp.min(score, axis=1, keepdims=True)
    eq = score == m
    k_iota = jax.lax.broadcasted_iota(
        jnp.int32, score.shape, 1).astype(jnp.float32)
    idx = jnp.min(jnp.where(eq, k_iota, float(2 * K)), axis=1)
    bins_ref[...] = idx.astype(jnp.int32)
    onehot = eq.astype(jnp.bfloat16)
    center = jax.lax.dot_general(
        onehot, chi_ref[...], (((1,), (0,)), ((), ())),
        preferred_element_type=jnp.float32)
    center_lo = jax.lax.dot_general(
        onehot, clo_ref[...], (((1,), (0,)), ((), ())),
        preferred_element_type=jnp.float32)
    res_ref[...] = a - (center + center_lo)


def kernel(action, centroids):
    B, T, D = action.shape
    K = centroids.shape[0]
    N = B * T
    a_flat = action.reshape(N, D)
    grid = (N // _R,)
    bins, res = pl.pallas_call(
        _vq_body,
        grid=grid,
        in_specs=[
            pl.BlockSpec((_R, D), lambda i: (i, 0)),
            pl.BlockSpec((K, D), lambda i: (0, 0)),
        ],
        out_specs=[
            pl.BlockSpec((_R,), lambda i: (i,)),
            pl.BlockSpec((_R, D), lambda i: (i, 0)),
        ],
        out_shape=[
            jax.ShapeDtypeStruct((N,), jnp.int32),
            jax.ShapeDtypeStruct((N, D), jnp.float32),
        ],
        scratch_shapes=[
            pltpu.VMEM((1, K), jnp.float32),
            pltpu.VMEM((K, D), jnp.bfloat16),
            pltpu.VMEM((K, D), jnp.bfloat16),
        ],
    )(a_flat, centroids)
    return (bins.reshape(B, T, 1).astype(jnp.int64),
            res.reshape(B, T, D))
